# baseline (device time: 191344 ns/iter reference)
import functools

import jax
import jax.numpy as jnp
from jax import lax
from jax.experimental import pallas as pl
from jax.experimental.pallas import tpu as pltpu

N_Y = 4
N_Z = 4
M, N = 8192, 1024
SLAB = 1024
PIECE = SLAB // N_Y


def kernel(x):
    my_x_o = lax.axis_index("x")
    my_z_o = lax.axis_index("z")
    g0 = my_x_o * (N_Z * SLAB) + my_z_o * SLAB
    x_slab = lax.dynamic_slice(x, (g0, 0), (SLAB, N)).astype(jnp.bfloat16)

    def body(x_ref, out_ref, comm_rs, comm_ag, slab_buf, comm_z,
             zslab_store, x_comm,
             send_rs, recv_rs, send_ag, recv_ag, send_z, recv_z,
             send_x, recv_x):
        my_x = lax.axis_index("x")
        my_y = lax.axis_index("y")
        my_z = lax.axis_index("z")
        y_left = (my_y - 1) % N_Y
        y_right = (my_y + 1) % N_Y
        z_left = (my_z - 1) % N_Z
        z_right = (my_z + 1) % N_Z
        other_x = 1 - my_x
        my_half = my_x * (N_Z * SLAB)
        their_half = other_x * (N_Z * SLAB)

        neighbors = [
            (my_x, y_left, my_z), (my_x, y_right, my_z),
            (my_x, my_y, z_left), (my_x, my_y, z_right),
            (other_x, my_y, my_z),
        ]
        barrier_sem = pltpu.get_barrier_semaphore()
        for dev in neighbors:
            pl.semaphore_signal(
                barrier_sem, inc=1,
                device_id=dev, device_id_type=pl.DeviceIdType.MESH,
            )
        pl.semaphore_wait(barrier_sem, len(neighbors))

        def chunk(c):
            return x_ref[pl.ds(c * PIECE, PIECE), :]

        comm_rs[0, :, :] = chunk(my_y)
        for s in range(N_Y - 1):
            rdma = pltpu.make_async_remote_copy(
                src_ref=comm_rs.at[s % 2],
                dst_ref=comm_rs.at[(s + 1) % 2],
                send_sem=send_rs.at[s % 2],
                recv_sem=recv_rs.at[(s + 1) % 2],
                device_id=(my_x, y_right, my_z),
                device_id_type=pl.DeviceIdType.MESH,
            )
            rdma.start()
            rdma.wait()
            c = (my_y - s - 1) % N_Y
            comm_rs[(s + 1) % 2, :, :] = comm_rs[(s + 1) % 2, :, :] + chunk(c)

        r = (my_y + 1) % N_Y
        slab_buf[pl.ds(r * PIECE, PIECE), :] = comm_rs[1, :, :]

        comm_ag[0, :, :] = comm_rs[1, :, :]
        for g in range(N_Y - 1):
            rdma = pltpu.make_async_remote_copy(
                src_ref=comm_ag.at[g % 2],
                dst_ref=comm_ag.at[(g + 1) % 2],
                send_sem=send_ag.at[g % 2],
                recv_sem=recv_ag.at[(g + 1) % 2],
                device_id=(my_x, y_right, my_z),
                device_id_type=pl.DeviceIdType.MESH,
            )
            rdma.start()
            rdma.wait()
            origin = (my_y - g) % N_Y
            slab_buf[pl.ds(origin * PIECE, PIECE), :] = comm_ag[(g + 1) % 2, :, :]

        out_ref[pl.ds(my_half + my_z * SLAB, SLAB), :] = slab_buf[:, :].astype(jnp.float32)

        def x_swap(k, src):
            rdma = pltpu.make_async_remote_copy(
                src_ref=src,
                dst_ref=x_comm.at[k % 2],
                send_sem=send_x.at[k % 2],
                recv_sem=recv_x.at[k % 2],
                device_id=(other_x, my_y, my_z),
                device_id_type=pl.DeviceIdType.MESH,
            )
            rdma.start()
            return rdma

        def z_hop(h):
            rdma = pltpu.make_async_remote_copy(
                src_ref=slab_buf if h == 0 else comm_z.at[h % 2],
                dst_ref=comm_z.at[(h + 1) % 2],
                send_sem=send_z.at[h % 2],
                recv_sem=recv_z.at[(h + 1) % 2],
                device_id=(my_x, my_y, z_right),
                device_id_type=pl.DeviceIdType.MESH,
            )
            rdma.start()
            return rdma

        sw = x_swap(0, slab_buf)
        zh = z_hop(0)
        for h in range(N_Z - 1):
            zh.wait()
            zslab_store[h, :, :] = comm_z[(h + 1) % 2, :, :]
            if h < N_Z - 2:
                zh = z_hop(h + 1)
            sw.wait_recv()
            out_ref[pl.ds(their_half + ((my_z - h) % N_Z) * SLAB, SLAB), :] = (
                x_comm[h % 2, :, :].astype(jnp.float32)
            )
            sw.wait_send()
            sw = x_swap(h + 1, zslab_store.at[h])
            origin_z = (my_z - h - 1) % N_Z
            out_ref[pl.ds(my_half + origin_z * SLAB, SLAB), :] = (
                zslab_store[h, :, :].astype(jnp.float32)
            )

        sw.wait_recv()
        out_ref[pl.ds(their_half + ((my_z - 3) % N_Z) * SLAB, SLAB), :] = (
            x_comm[3 % 2, :, :].astype(jnp.float32)
        )
        sw.wait_send()

        @functools.partial(
            pl.run_scoped, second_barrier=pltpu.SemaphoreType.REGULAR
        )
        def _(second_barrier):
            for dev in neighbors:
                pl.semaphore_signal(
                    second_barrier, inc=1,
                    device_id=dev, device_id_type=pl.DeviceIdType.MESH,
                )
            pl.semaphore_wait(second_barrier, len(neighbors))

    out = pl.pallas_call(
        body,
        out_shape=jax.ShapeDtypeStruct((M, N), jnp.float32),
        in_specs=[pl.BlockSpec(memory_space=pltpu.VMEM)],
        out_specs=pl.BlockSpec(memory_space=pltpu.VMEM),
        scratch_shapes=[
            pltpu.VMEM((2, PIECE, N), jnp.bfloat16),
            pltpu.VMEM((2, PIECE, N), jnp.bfloat16),
            pltpu.VMEM((SLAB, N), jnp.bfloat16),
            pltpu.VMEM((2, SLAB, N), jnp.bfloat16),
            pltpu.VMEM((N_Z - 1, SLAB, N), jnp.bfloat16),
            pltpu.VMEM((2, SLAB, N), jnp.bfloat16),
            pltpu.SemaphoreType.DMA((2,)),
            pltpu.SemaphoreType.DMA((2,)),
            pltpu.SemaphoreType.DMA((2,)),
            pltpu.SemaphoreType.DMA((2,)),
            pltpu.SemaphoreType.DMA((2,)),
            pltpu.SemaphoreType.DMA((2,)),
            pltpu.SemaphoreType.DMA((2,)),
            pltpu.SemaphoreType.DMA((2,)),
        ],
        compiler_params=pltpu.CompilerParams(
            collective_id=0,
            vmem_limit_bytes=100 * 1024 * 1024,
        ),
    )(x_slab)
    return out


# device time: 180905 ns/iter; 1.0577x vs baseline; 1.0577x over previous
import functools

import jax
import jax.numpy as jnp
from jax import lax
from jax.experimental import pallas as pl
from jax.experimental.pallas import tpu as pltpu

N_Y = 4
N_Z = 4
M, N = 8192, 1024
SLAB = 1024
PIECE = SLAB // N_Y


def kernel(x):
    my_x_o = lax.axis_index("x")
    my_z_o = lax.axis_index("z")
    g0 = my_x_o * (N_Z * SLAB) + my_z_o * SLAB
    x_slab = lax.dynamic_slice(x, (g0, 0), (SLAB, N)).astype(jnp.bfloat16)

    def body(x_ref, out_ref, comm_rs, comm_ag, slab_buf, comm_z,
             zslab_store, x_comm,
             send_rs, recv_rs, send_ag, recv_ag, send_z, recv_z,
             send_x, recv_x):
        my_x = lax.axis_index("x")
        my_y = lax.axis_index("y")
        my_z = lax.axis_index("z")
        y_left = (my_y - 1) % N_Y
        y_right = (my_y + 1) % N_Y
        z_left = (my_z - 1) % N_Z
        z_right = (my_z + 1) % N_Z
        other_x = 1 - my_x
        my_half = my_x * (N_Z * SLAB)
        their_half = other_x * (N_Z * SLAB)

        neighbors = [
            (my_x, y_left, my_z), (my_x, y_right, my_z),
            (my_x, my_y, z_left), (my_x, my_y, z_right),
            (other_x, my_y, my_z),
        ]
        barrier_sem = pltpu.get_barrier_semaphore()
        for dev in neighbors:
            pl.semaphore_signal(
                barrier_sem, inc=1,
                device_id=dev, device_id_type=pl.DeviceIdType.MESH,
            )
        pl.semaphore_wait(barrier_sem, len(neighbors))

        def chunk(c):
            return x_ref[pl.ds(c * PIECE, PIECE), :]

        comm_rs[0, :, :] = chunk(my_y)
        for s in range(N_Y - 1):
            rdma = pltpu.make_async_remote_copy(
                src_ref=comm_rs.at[s % 2],
                dst_ref=comm_rs.at[(s + 1) % 2],
                send_sem=send_rs.at[s % 2],
                recv_sem=recv_rs.at[(s + 1) % 2],
                device_id=(my_x, y_right, my_z),
                device_id_type=pl.DeviceIdType.MESH,
            )
            rdma.start()
            rdma.wait()
            c = (my_y - s - 1) % N_Y
            comm_rs[(s + 1) % 2, :, :] = comm_rs[(s + 1) % 2, :, :] + chunk(c)

        r = (my_y + 1) % N_Y
        slab_buf[pl.ds(r * PIECE, PIECE), :] = comm_rs[1, :, :]

        comm_ag[0, :, :] = comm_rs[1, :, :]
        for g in range(N_Y - 1):
            rdma = pltpu.make_async_remote_copy(
                src_ref=comm_ag.at[g % 2],
                dst_ref=comm_ag.at[(g + 1) % 2],
                send_sem=send_ag.at[g % 2],
                recv_sem=recv_ag.at[(g + 1) % 2],
                device_id=(my_x, y_right, my_z),
                device_id_type=pl.DeviceIdType.MESH,
            )
            rdma.start()
            rdma.wait()
            origin = (my_y - g) % N_Y
            slab_buf[pl.ds(origin * PIECE, PIECE), :] = comm_ag[(g + 1) % 2, :, :]

        out_ref[pl.ds(my_half + my_z * SLAB, SLAB), :] = slab_buf[:, :]

        def x_swap(k, src):
            rdma = pltpu.make_async_remote_copy(
                src_ref=src,
                dst_ref=x_comm.at[k % 2],
                send_sem=send_x.at[k % 2],
                recv_sem=recv_x.at[k % 2],
                device_id=(other_x, my_y, my_z),
                device_id_type=pl.DeviceIdType.MESH,
            )
            rdma.start()
            return rdma

        def z_hop(h):
            rdma = pltpu.make_async_remote_copy(
                src_ref=slab_buf if h == 0 else comm_z.at[h % 2],
                dst_ref=comm_z.at[(h + 1) % 2],
                send_sem=send_z.at[h % 2],
                recv_sem=recv_z.at[(h + 1) % 2],
                device_id=(my_x, my_y, z_right),
                device_id_type=pl.DeviceIdType.MESH,
            )
            rdma.start()
            return rdma

        sw = x_swap(0, slab_buf)
        zh = z_hop(0)
        for h in range(N_Z - 1):
            zh.wait()
            zslab_store[h, :, :] = comm_z[(h + 1) % 2, :, :]
            if h < N_Z - 2:
                zh = z_hop(h + 1)
            sw.wait_recv()
            out_ref[pl.ds(their_half + ((my_z - h) % N_Z) * SLAB, SLAB), :] = (
                x_comm[h % 2, :, :]
            )
            sw.wait_send()
            sw = x_swap(h + 1, zslab_store.at[h])
            origin_z = (my_z - h - 1) % N_Z
            out_ref[pl.ds(my_half + origin_z * SLAB, SLAB), :] = (
                zslab_store[h, :, :]
            )

        sw.wait_recv()
        out_ref[pl.ds(their_half + ((my_z - 3) % N_Z) * SLAB, SLAB), :] = (
            x_comm[3 % 2, :, :]
        )
        sw.wait_send()

        @functools.partial(
            pl.run_scoped, second_barrier=pltpu.SemaphoreType.REGULAR
        )
        def _(second_barrier):
            for dev in neighbors:
                pl.semaphore_signal(
                    second_barrier, inc=1,
                    device_id=dev, device_id_type=pl.DeviceIdType.MESH,
                )
            pl.semaphore_wait(second_barrier, len(neighbors))

    out = pl.pallas_call(
        body,
        out_shape=jax.ShapeDtypeStruct((M, N), jnp.bfloat16),
        in_specs=[pl.BlockSpec(memory_space=pltpu.VMEM)],
        out_specs=pl.BlockSpec(memory_space=pltpu.VMEM),
        scratch_shapes=[
            pltpu.VMEM((2, PIECE, N), jnp.bfloat16),
            pltpu.VMEM((2, PIECE, N), jnp.bfloat16),
            pltpu.VMEM((SLAB, N), jnp.bfloat16),
            pltpu.VMEM((2, SLAB, N), jnp.bfloat16),
            pltpu.VMEM((N_Z - 1, SLAB, N), jnp.bfloat16),
            pltpu.VMEM((2, SLAB, N), jnp.bfloat16),
            pltpu.SemaphoreType.DMA((2,)),
            pltpu.SemaphoreType.DMA((2,)),
            pltpu.SemaphoreType.DMA((2,)),
            pltpu.SemaphoreType.DMA((2,)),
            pltpu.SemaphoreType.DMA((2,)),
            pltpu.SemaphoreType.DMA((2,)),
            pltpu.SemaphoreType.DMA((2,)),
            pltpu.SemaphoreType.DMA((2,)),
        ],
        compiler_params=pltpu.CompilerParams(
            collective_id=0,
            vmem_limit_bytes=100 * 1024 * 1024,
        ),
    )(x_slab)
    return out.astype(jnp.float32)


# device time: 178493 ns/iter; 1.0720x vs baseline; 1.0135x over previous
import functools

import jax
import jax.numpy as jnp
from jax import lax
from jax.experimental import pallas as pl
from jax.experimental.pallas import tpu as pltpu

N_Y = 4
N_Z = 4
M, N = 8192, 1024
SLAB = 1024
PIECE = SLAB // N_Y
N_HOP = N_Z - 1


def kernel(x):
    my_x_o = lax.axis_index("x")
    my_z_o = lax.axis_index("z")
    g0 = my_x_o * (N_Z * SLAB) + my_z_o * SLAB
    x_slab = lax.dynamic_slice(x, (g0, 0), (SLAB, N)).astype(jnp.bfloat16)

    def body(x_ref, out_ref, comm_rs, comm_ag, piece_store, comm_zp,
             zpiece_store, x_comm,
             send_rs, recv_rs, send_ag, recv_ag, send_z, recv_z,
             send_x, recv_x):
        my_x = lax.axis_index("x")
        my_y = lax.axis_index("y")
        my_z = lax.axis_index("z")
        y_right = (my_y + 1) % N_Y
        other_x = 1 - my_x
        my_half = my_x * (N_Z * SLAB)
        their_half = other_x * (N_Z * SLAB)

        neighbors = [
            (my_x, (my_y - 1) % N_Y, my_z), (my_x, y_right, my_z),
            (my_x, my_y, (my_z - 1) % N_Z), (my_x, my_y, (my_z + 1) % N_Z),
            (other_x, my_y, my_z),
        ]
        barrier_sem = pltpu.get_barrier_semaphore()
        for dev in neighbors:
            pl.semaphore_signal(
                barrier_sem, inc=1,
                device_id=dev, device_id_type=pl.DeviceIdType.MESH,
            )
        pl.semaphore_wait(barrier_sem, len(neighbors))

        def pid(k):
            return (my_y + 1 - k) % N_Y

        def y_hop(i, comm, send, recv):
            rdma = pltpu.make_async_remote_copy(
                src_ref=comm.at[i % 2],
                dst_ref=comm.at[(i + 1) % 2],
                send_sem=send.at[i % 2],
                recv_sem=recv.at[(i + 1) % 2],
                device_id=(my_x, y_right, my_z),
                device_id_type=pl.DeviceIdType.MESH,
            )
            rdma.start()
            return rdma

        def z_start(k, h):
            rdma = pltpu.make_async_remote_copy(
                src_ref=(piece_store.at[k] if h == 0
                         else comm_zp.at[k, (h - 1) % 2]),
                dst_ref=comm_zp.at[k, h % 2],
                send_sem=send_z.at[k],
                recv_sem=recv_z.at[k],
                device_id=(my_x, my_y, (my_z + 1) % N_Z),
                device_id_type=pl.DeviceIdType.MESH,
            )
            rdma.start()
            return rdma

        def z_finish(zrdma, k, h):
            zrdma.wait()
            zpiece_store[k, h, :, :] = comm_zp[k, h % 2, :, :]
            oz = (my_z - 1 - h) % N_Z
            out_ref[pl.ds(my_half + oz * SLAB + pid(k) * PIECE, PIECE), :] = (
                zpiece_store[k, h, :, :]
            )
            return z_start(k, h + 1) if h + 1 < N_HOP else None

        def x_start(j, tag):
            src = (piece_store.at[tag[1]] if tag[0] == "own"
                   else zpiece_store.at[tag[1], tag[2]])
            rdma = pltpu.make_async_remote_copy(
                src_ref=src,
                dst_ref=x_comm.at[j % 2],
                send_sem=send_x.at[j % 2],
                recv_sem=recv_x.at[j % 2],
                device_id=(other_x, my_y, my_z),
                device_id_type=pl.DeviceIdType.MESH,
            )
            rdma.start()
            return rdma

        def x_finish(xrdma, j, tag):
            xrdma.wait_recv()
            oz = my_z if tag[0] == "own" else (my_z - 1 - tag[2]) % N_Z
            out_ref[
                pl.ds(their_half + oz * SLAB + pid(tag[1]) * PIECE, PIECE), :
            ] = x_comm[j % 2, :, :]
            xrdma.wait_send()

        comm_rs[0, :, :] = x_ref[pl.ds(my_y * PIECE, PIECE), :]
        for s in range(N_Y - 1):
            rdma = y_hop(s, comm_rs, send_rs, recv_rs)
            rdma.wait()
            c = (my_y - s - 1) % N_Y
            comm_rs[(s + 1) % 2, :, :] = (
                comm_rs[(s + 1) % 2, :, :] + x_ref[pl.ds(c * PIECE, PIECE), :]
            )

        piece_store[0, :, :] = comm_rs[1, :, :]
        out_ref[pl.ds(my_half + my_z * SLAB + pid(0) * PIECE, PIECE), :] = (
            piece_store[0, :, :]
        )

        comm_ag[0, :, :] = comm_rs[1, :, :]
        z_live = {0: (z_start(0, 0), 0)}
        state = {"x": (x_start(0, ("own", 0)), 0, ("own", 0)), "j": 0}

        def x_adv(tag):
            x_finish(*state["x"])
            state["j"] += 1
            state["x"] = (x_start(state["j"], tag), state["j"], tag)

        def z_adv(k):
            zrdma, h = z_live[k]
            nxt = z_finish(zrdma, k, h)
            if nxt is not None:
                z_live[k] = (nxt, h + 1)
            else:
                del z_live[k]
            return k, h

        for k in range(1, N_Y):
            ag = y_hop(k - 1, comm_ag, send_ag, recv_ag)
            ag.wait()
            piece_store[k, :, :] = comm_ag[k % 2, :, :]
            out_ref[
                pl.ds(my_half + my_z * SLAB + pid(k) * PIECE, PIECE), :
            ] = piece_store[k, :, :]
            z_live[k] = (z_start(k, 0), 0)
            x_adv(("own", k))
            for kk in range(k - 1, -1, -1):
                if kk in z_live:
                    kz, hz = z_adv(kk)
                    x_adv(("z", kz, hz))

        while z_live:
            for kk in sorted(z_live, key=lambda q: (z_live[q][1], -q)):
                kz, hz = z_adv(kk)
                x_adv(("z", kz, hz))

        x_finish(*state["x"])

        @functools.partial(
            pl.run_scoped, second_barrier=pltpu.SemaphoreType.REGULAR
        )
        def _(second_barrier):
            for dev in neighbors:
                pl.semaphore_signal(
                    second_barrier, inc=1,
                    device_id=dev, device_id_type=pl.DeviceIdType.MESH,
                )
            pl.semaphore_wait(second_barrier, len(neighbors))

    out = pl.pallas_call(
        body,
        out_shape=jax.ShapeDtypeStruct((M, N), jnp.bfloat16),
        in_specs=[pl.BlockSpec(memory_space=pltpu.VMEM)],
        out_specs=pl.BlockSpec(memory_space=pltpu.VMEM),
        scratch_shapes=[
            pltpu.VMEM((2, PIECE, N), jnp.bfloat16),
            pltpu.VMEM((2, PIECE, N), jnp.bfloat16),
            pltpu.VMEM((N_Y, PIECE, N), jnp.bfloat16),
            pltpu.VMEM((N_Y, 2, PIECE, N), jnp.bfloat16),
            pltpu.VMEM((N_Y, N_HOP, PIECE, N), jnp.bfloat16),
            pltpu.VMEM((2, PIECE, N), jnp.bfloat16),
            pltpu.SemaphoreType.DMA((2,)),
            pltpu.SemaphoreType.DMA((2,)),
            pltpu.SemaphoreType.DMA((2,)),
            pltpu.SemaphoreType.DMA((2,)),
            pltpu.SemaphoreType.DMA((N_Y,)),
            pltpu.SemaphoreType.DMA((N_Y,)),
            pltpu.SemaphoreType.DMA((2,)),
            pltpu.SemaphoreType.DMA((2,)),
        ],
        compiler_params=pltpu.CompilerParams(
            collective_id=0,
            vmem_limit_bytes=100 * 1024 * 1024,
        ),
    )(x_slab)
    return out.astype(jnp.float32)


# device time: 154550 ns/iter; 1.2381x vs baseline; 1.1549x over previous
import functools

import jax
import jax.numpy as jnp
from jax import lax
from jax.experimental import pallas as pl
from jax.experimental.pallas import tpu as pltpu

N_Y = 4
N_Z = 4
M, N = 8192, 1024
SLAB = 1024
PIECE = SLAB // N_Y
N_HOP = N_Z - 1


def kernel(x):
    my_x_o = lax.axis_index("x")
    my_z_o = lax.axis_index("z")
    g0 = my_x_o * (N_Z * SLAB) + my_z_o * SLAB
    x_slab = lax.dynamic_slice(x, (g0, 0), (SLAB, N)).astype(jnp.bfloat16)

    def body(x_ref, out_ref, comm_rs, comm_ag, piece_store, comm_zp,
             zpiece_store, x_comm,
             send_rs, recv_rs, send_ag, recv_ag, send_z, recv_z,
             send_x, recv_x):
        my_x = lax.axis_index("x")
        my_y = lax.axis_index("y")
        my_z = lax.axis_index("z")
        y_right = (my_y + 1) % N_Y
        other_x = 1 - my_x
        my_half = my_x * (N_Z * SLAB)
        their_half = other_x * (N_Z * SLAB)

        neighbors = [
            (my_x, (my_y - 1) % N_Y, my_z), (my_x, y_right, my_z),
            (my_x, my_y, (my_z - 1) % N_Z), (my_x, my_y, (my_z + 1) % N_Z),
            (other_x, my_y, my_z),
        ]
        barrier_sem = pltpu.get_barrier_semaphore()
        for dev in neighbors:
            pl.semaphore_signal(
                barrier_sem, inc=1,
                device_id=dev, device_id_type=pl.DeviceIdType.MESH,
            )
        pl.semaphore_wait(barrier_sem, len(neighbors))

        def pid(k):
            return (my_y + 1 - k) % N_Y

        def y_hop(i, comm, send, recv):
            rdma = pltpu.make_async_remote_copy(
                src_ref=comm.at[i % 2],
                dst_ref=comm.at[(i + 1) % 2],
                send_sem=send.at[i % 2],
                recv_sem=recv.at[(i + 1) % 2],
                device_id=(my_x, y_right, my_z),
                device_id_type=pl.DeviceIdType.MESH,
            )
            rdma.start()
            return rdma

        def z_start(k, h):
            rdma = pltpu.make_async_remote_copy(
                src_ref=(piece_store.at[k] if h == 0
                         else comm_zp.at[k, (h - 1) % 2]),
                dst_ref=comm_zp.at[k, h % 2],
                send_sem=send_z.at[k],
                recv_sem=recv_z.at[k],
                device_id=(my_x, my_y, (my_z + 1) % N_Z),
                device_id_type=pl.DeviceIdType.MESH,
            )
            rdma.start()
            return rdma

        def z_finish(zrdma, k, h):
            zrdma.wait()
            zpiece_store[k, h, :, :] = comm_zp[k, h % 2, :, :]
            oz = (my_z - 1 - h) % N_Z
            out_ref[pl.ds(my_half + oz * SLAB + pid(k) * PIECE, PIECE), :] = (
                zpiece_store[k, h, :, :]
            )
            return z_start(k, h + 1) if h + 1 < N_HOP else None

        def x_start(j, tag):
            src = (piece_store.at[tag[1]] if tag[0] == "own"
                   else zpiece_store.at[tag[1], tag[2]])
            rdma = pltpu.make_async_remote_copy(
                src_ref=src,
                dst_ref=x_comm.at[j % 4],
                send_sem=send_x.at[j % 4],
                recv_sem=recv_x.at[j % 4],
                device_id=(other_x, my_y, my_z),
                device_id_type=pl.DeviceIdType.MESH,
            )
            rdma.start()
            return rdma

        def x_finish(xrdma, j, tag):
            xrdma.wait_recv()
            oz = my_z if tag[0] == "own" else (my_z - 1 - tag[2]) % N_Z
            out_ref[
                pl.ds(their_half + oz * SLAB + pid(tag[1]) * PIECE, PIECE), :
            ] = x_comm[j % 4, :, :]
            xrdma.wait_send()

        comm_rs[0, :, :] = x_ref[pl.ds(my_y * PIECE, PIECE), :]
        for s in range(N_Y - 1):
            rdma = y_hop(s, comm_rs, send_rs, recv_rs)
            rdma.wait()
            c = (my_y - s - 1) % N_Y
            comm_rs[(s + 1) % 2, :, :] = (
                comm_rs[(s + 1) % 2, :, :] + x_ref[pl.ds(c * PIECE, PIECE), :]
            )

        piece_store[0, :, :] = comm_rs[1, :, :]
        out_ref[pl.ds(my_half + my_z * SLAB + pid(0) * PIECE, PIECE), :] = (
            piece_store[0, :, :]
        )

        comm_ag[0, :, :] = comm_rs[1, :, :]
        z_live = {0: (z_start(0, 0), 0)}
        state = {"q": [(x_start(0, ("own", 0)), 0, ("own", 0))], "j": 0}

        def x_adv(tag):
            if len(state["q"]) == 2:
                x_finish(*state["q"].pop(0))
            state["j"] += 1
            state["q"].append((x_start(state["j"], tag), state["j"], tag))

        def z_adv(k):
            zrdma, h = z_live[k]
            nxt = z_finish(zrdma, k, h)
            if nxt is not None:
                z_live[k] = (nxt, h + 1)
            else:
                del z_live[k]
            return k, h

        for k in range(1, N_Y):
            ag = y_hop(k - 1, comm_ag, send_ag, recv_ag)
            ag.wait()
            piece_store[k, :, :] = comm_ag[k % 2, :, :]
            out_ref[
                pl.ds(my_half + my_z * SLAB + pid(k) * PIECE, PIECE), :
            ] = piece_store[k, :, :]
            z_live[k] = (z_start(k, 0), 0)
            x_adv(("own", k))
            for kk in range(k - 1, -1, -1):
                if kk in z_live:
                    kz, hz = z_adv(kk)
                    x_adv(("z", kz, hz))

        while z_live:
            for kk in sorted(z_live, key=lambda q: (z_live[q][1], -q)):
                kz, hz = z_adv(kk)
                x_adv(("z", kz, hz))

        for entry in state["q"]:
            x_finish(*entry)

        @functools.partial(
            pl.run_scoped, second_barrier=pltpu.SemaphoreType.REGULAR
        )
        def _(second_barrier):
            for dev in neighbors:
                pl.semaphore_signal(
                    second_barrier, inc=1,
                    device_id=dev, device_id_type=pl.DeviceIdType.MESH,
                )
            pl.semaphore_wait(second_barrier, len(neighbors))

    out = pl.pallas_call(
        body,
        out_shape=jax.ShapeDtypeStruct((M, N), jnp.bfloat16),
        in_specs=[pl.BlockSpec(memory_space=pltpu.VMEM)],
        out_specs=pl.BlockSpec(memory_space=pltpu.VMEM),
        scratch_shapes=[
            pltpu.VMEM((2, PIECE, N), jnp.bfloat16),
            pltpu.VMEM((2, PIECE, N), jnp.bfloat16),
            pltpu.VMEM((N_Y, PIECE, N), jnp.bfloat16),
            pltpu.VMEM((N_Y, 2, PIECE, N), jnp.bfloat16),
            pltpu.VMEM((N_Y, N_HOP, PIECE, N), jnp.bfloat16),
            pltpu.VMEM((4, PIECE, N), jnp.bfloat16),
            pltpu.SemaphoreType.DMA((2,)),
            pltpu.SemaphoreType.DMA((2,)),
            pltpu.SemaphoreType.DMA((2,)),
            pltpu.SemaphoreType.DMA((2,)),
            pltpu.SemaphoreType.DMA((N_Y,)),
            pltpu.SemaphoreType.DMA((N_Y,)),
            pltpu.SemaphoreType.DMA((4,)),
            pltpu.SemaphoreType.DMA((4,)),
        ],
        compiler_params=pltpu.CompilerParams(
            collective_id=0,
            vmem_limit_bytes=100 * 1024 * 1024,
        ),
    )(x_slab)
    return out.astype(jnp.float32)


# device time: 154541 ns/iter; 1.2381x vs baseline; 1.0001x over previous
import functools

import jax
import jax.numpy as jnp
from jax import lax
from jax.experimental import pallas as pl
from jax.experimental.pallas import tpu as pltpu

N_Y = 4
N_Z = 4
M, N = 8192, 1024
SLAB = 1024
PIECE = SLAB // N_Y
N_HOP = N_Z - 1


def kernel(x):
    my_x_o = lax.axis_index("x")
    my_z_o = lax.axis_index("z")
    g0 = my_x_o * (N_Z * SLAB) + my_z_o * SLAB
    x_slab = lax.dynamic_slice(x, (g0, 0), (SLAB, N)).astype(jnp.bfloat16)

    def body(x_ref, out_ref, comm_rs, comm_ag, piece_store, comm_zp,
             zpiece_store, x_comm,
             send_rs, recv_rs, send_ag, recv_ag, send_z, recv_z,
             send_x, recv_x):
        my_x = lax.axis_index("x")
        my_y = lax.axis_index("y")
        my_z = lax.axis_index("z")
        y_right = (my_y + 1) % N_Y
        other_x = 1 - my_x
        my_half = my_x * (N_Z * SLAB)
        their_half = other_x * (N_Z * SLAB)

        neighbors = [
            (my_x, (my_y - 1) % N_Y, my_z), (my_x, y_right, my_z),
            (my_x, my_y, (my_z - 1) % N_Z), (my_x, my_y, (my_z + 1) % N_Z),
            (other_x, my_y, my_z),
        ]
        barrier_sem = pltpu.get_barrier_semaphore()
        for dev in neighbors:
            pl.semaphore_signal(
                barrier_sem, inc=1,
                device_id=dev, device_id_type=pl.DeviceIdType.MESH,
            )
        pl.semaphore_wait(barrier_sem, len(neighbors))

        def pid(k):
            return (my_y + 1 - k) % N_Y

        def y_hop(i, comm, send, recv):
            rdma = pltpu.make_async_remote_copy(
                src_ref=comm.at[i % 2],
                dst_ref=comm.at[(i + 1) % 2],
                send_sem=send.at[i % 2],
                recv_sem=recv.at[(i + 1) % 2],
                device_id=(my_x, y_right, my_z),
                device_id_type=pl.DeviceIdType.MESH,
            )
            rdma.start()
            return rdma

        def z_start(k, h):
            rdma = pltpu.make_async_remote_copy(
                src_ref=(piece_store.at[k] if h == 0
                         else comm_zp.at[k, (h - 1) % 2]),
                dst_ref=comm_zp.at[k, h % 2],
                send_sem=send_z.at[k],
                recv_sem=recv_z.at[k],
                device_id=(my_x, my_y, (my_z + 1) % N_Z),
                device_id_type=pl.DeviceIdType.MESH,
            )
            rdma.start()
            return rdma

        def z_finish(zrdma, k, h):
            zrdma.wait()
            zpiece_store[k, h, :, :] = comm_zp[k, h % 2, :, :]
            oz = (my_z - 1 - h) % N_Z
            out_ref[pl.ds(my_half + oz * SLAB + pid(k) * PIECE, PIECE), :] = (
                zpiece_store[k, h, :, :]
            )
            return z_start(k, h + 1) if h + 1 < N_HOP else None

        def x_start(j, tag):
            src = (piece_store.at[tag[1]] if tag[0] == "own"
                   else zpiece_store.at[tag[1], tag[2]])
            rdma = pltpu.make_async_remote_copy(
                src_ref=src,
                dst_ref=x_comm.at[j % 6],
                send_sem=send_x.at[j % 6],
                recv_sem=recv_x.at[j % 6],
                device_id=(other_x, my_y, my_z),
                device_id_type=pl.DeviceIdType.MESH,
            )
            rdma.start()
            return rdma

        def x_finish(xrdma, j, tag):
            xrdma.wait_recv()
            oz = my_z if tag[0] == "own" else (my_z - 1 - tag[2]) % N_Z
            out_ref[
                pl.ds(their_half + oz * SLAB + pid(tag[1]) * PIECE, PIECE), :
            ] = x_comm[j % 6, :, :]
            xrdma.wait_send()

        comm_rs[0, :, :] = x_ref[pl.ds(my_y * PIECE, PIECE), :]
        for s in range(N_Y - 1):
            rdma = y_hop(s, comm_rs, send_rs, recv_rs)
            rdma.wait()
            c = (my_y - s - 1) % N_Y
            comm_rs[(s + 1) % 2, :, :] = (
                comm_rs[(s + 1) % 2, :, :] + x_ref[pl.ds(c * PIECE, PIECE), :]
            )

        piece_store[0, :, :] = comm_rs[1, :, :]
        out_ref[pl.ds(my_half + my_z * SLAB + pid(0) * PIECE, PIECE), :] = (
            piece_store[0, :, :]
        )

        comm_ag[0, :, :] = comm_rs[1, :, :]
        z_live = {0: (z_start(0, 0), 0)}
        state = {"q": [(x_start(0, ("own", 0)), 0, ("own", 0))], "j": 0}

        def x_adv(tag):
            if len(state["q"]) == 3:
                x_finish(*state["q"].pop(0))
            state["j"] += 1
            state["q"].append((x_start(state["j"], tag), state["j"], tag))

        def z_adv(k):
            zrdma, h = z_live[k]
            nxt = z_finish(zrdma, k, h)
            if nxt is not None:
                z_live[k] = (nxt, h + 1)
            else:
                del z_live[k]
            return k, h

        for k in range(1, N_Y):
            ag = y_hop(k - 1, comm_ag, send_ag, recv_ag)
            ag.wait()
            piece_store[k, :, :] = comm_ag[k % 2, :, :]
            out_ref[
                pl.ds(my_half + my_z * SLAB + pid(k) * PIECE, PIECE), :
            ] = piece_store[k, :, :]
            z_live[k] = (z_start(k, 0), 0)
            x_adv(("own", k))
            for kk in range(k - 1, -1, -1):
                if kk in z_live:
                    kz, hz = z_adv(kk)
                    x_adv(("z", kz, hz))

        while z_live:
            for kk in sorted(z_live, key=lambda q: (z_live[q][1], -q)):
                kz, hz = z_adv(kk)
                x_adv(("z", kz, hz))

        for entry in state["q"]:
            x_finish(*entry)

        @functools.partial(
            pl.run_scoped, second_barrier=pltpu.SemaphoreType.REGULAR
        )
        def _(second_barrier):
            for dev in neighbors:
                pl.semaphore_signal(
                    second_barrier, inc=1,
                    device_id=dev, device_id_type=pl.DeviceIdType.MESH,
                )
            pl.semaphore_wait(second_barrier, len(neighbors))

    out = pl.pallas_call(
        body,
        out_shape=jax.ShapeDtypeStruct((M, N), jnp.bfloat16),
        in_specs=[pl.BlockSpec(memory_space=pltpu.VMEM)],
        out_specs=pl.BlockSpec(memory_space=pltpu.VMEM),
        scratch_shapes=[
            pltpu.VMEM((2, PIECE, N), jnp.bfloat16),
            pltpu.VMEM((2, PIECE, N), jnp.bfloat16),
            pltpu.VMEM((N_Y, PIECE, N), jnp.bfloat16),
            pltpu.VMEM((N_Y, 2, PIECE, N), jnp.bfloat16),
            pltpu.VMEM((N_Y, N_HOP, PIECE, N), jnp.bfloat16),
            pltpu.VMEM((6, PIECE, N), jnp.bfloat16),
            pltpu.SemaphoreType.DMA((2,)),
            pltpu.SemaphoreType.DMA((2,)),
            pltpu.SemaphoreType.DMA((2,)),
            pltpu.SemaphoreType.DMA((2,)),
            pltpu.SemaphoreType.DMA((N_Y,)),
            pltpu.SemaphoreType.DMA((N_Y,)),
            pltpu.SemaphoreType.DMA((6,)),
            pltpu.SemaphoreType.DMA((6,)),
        ],
        compiler_params=pltpu.CompilerParams(
            collective_id=0,
            vmem_limit_bytes=100 * 1024 * 1024,
        ),
    )(x_slab)
    return out.astype(jnp.float32)
